# baseline (speedup 1.0000x reference)
import functools

import jax
import jax.numpy as jnp
import numpy as np
from jax import lax
from jax.experimental import pallas as pl
from jax.experimental.pallas import tpu as pltpu

N_ATOMS = 10000
NP = 10240
N_EDGES = 160000
N_SYS = 16
SPH = 64
SPHSZ = 9
PP = 5
C2 = 128
NDB = 256
EDGE_CH = 64
CUTOFF = 5.0
NUM_LAYERS = 2
MAX_ELEM = 100

BN = 1024
BE = 800

INTERPRET = False


def _silu(x):
    return x * jax.nn.sigmoid(x)


def _node_prep_body(anum_ref, batch_ref, charge_ref, spin_ref, freqs_ref,
                    mixw_ref, mixb_ref, sphere_ref, stw_ref,
                    x0_ref, sysn_ref, nst_ref):
    chg = charge_ref[...]
    spn = spin_ref[...]
    freqs = freqs_ref[...]
    angc = chg * freqs
    angs = spn * freqs
    mw = mixw_ref[...]
    csd = (jnp.dot(jnp.sin(angc), mw[0:32], preferred_element_type=jnp.float32)
           + jnp.dot(jnp.cos(angc), mw[32:64], preferred_element_type=jnp.float32)
           + jnp.dot(jnp.sin(angs), mw[64:96], preferred_element_type=jnp.float32)
           + jnp.dot(jnp.cos(angs), mw[96:128], preferred_element_type=jnp.float32))
    csd = _silu(csd + mixb_ref[...])
    anum = anum_ref[...]
    oh_a = (anum == lax.broadcasted_iota(jnp.int32, (BN, 128), 1)).astype(jnp.float32)
    batch = batch_ref[...]
    oh_b = (batch == lax.broadcasted_iota(jnp.int32, (BN, 16), 1)).astype(jnp.float32)
    atom_emb = jnp.dot(oh_a, sphere_ref[...], preferred_element_type=jnp.float32)
    sys_node = jnp.dot(oh_b, csd, preferred_element_type=jnp.float32)
    x0_ref[...] = atom_emb + sys_node
    sysn_ref[...] = sys_node
    nst_ref[...] = jnp.dot(oh_a, stw_ref[...], preferred_element_type=jnp.float32)


def _node_prep(anum2, batch2, charge2, spin2, freqs2, mixw, mixb, sphere_p, stw):
    grid = (NP // BN,)
    full = lambda i: (0, 0)
    return pl.pallas_call(
        _node_prep_body,
        grid=grid,
        in_specs=[
            pl.BlockSpec((BN, 1), lambda i: (i, 0)),
            pl.BlockSpec((BN, 1), lambda i: (i, 0)),
            pl.BlockSpec((N_SYS, 1), full),
            pl.BlockSpec((N_SYS, 1), full),
            pl.BlockSpec((1, 32), full),
            pl.BlockSpec((128, SPH), full),
            pl.BlockSpec((1, SPH), full),
            pl.BlockSpec((128, SPH), full),
            pl.BlockSpec((128, C2), full),
        ],
        out_specs=[
            pl.BlockSpec((BN, SPH), lambda i: (i, 0)),
            pl.BlockSpec((BN, SPH), lambda i: (i, 0)),
            pl.BlockSpec((BN, C2), lambda i: (i, 0)),
        ],
        out_shape=[
            jax.ShapeDtypeStruct((NP, SPH), jnp.float32),
            jax.ShapeDtypeStruct((NP, SPH), jnp.float32),
            jax.ShapeDtypeStruct((NP, C2), jnp.float32),
        ],
        interpret=INTERPRET,
    )(anum2, batch2, charge2, spin2, freqs2, mixw, mixb, sphere_p, stw)


def _edge_prep_body(ev_ref, g1_ref, g2_ref, w1d_ref, b1_ref,
                    w2_ref, b2_ref, w3p_ref, gw_ref,
                    msg0_ref, gates_ref, wig_ref):
    ev = ev_ref[...]
    vx, vy, vz = ev[:, 0:1], ev[:, 1:2], ev[:, 2:3]
    d2 = vx * vx + vy * vy + vz * vz
    dist = jnp.sqrt(d2)
    u = dist * (1.0 / CUTOFF)
    u2 = u * u
    u4 = u2 * u2
    u5 = u4 * u
    env = 1.0 + (-21.0) * u5 + 35.0 * u5 * u + (-15.0) * u5 * u2
    env = jnp.where(u < 1.0, env, 0.0)
    step = CUTOFF / (NDB - 1)
    coeff = -0.5 / (2.0 * step) ** 2
    offs = lax.broadcasted_iota(jnp.int32, (BE, NDB), 1).astype(jnp.float32) * step
    diff = dist - offs
    demb = jnp.exp(coeff * diff * diff)
    h = jnp.dot(demb, w1d_ref[...], preferred_element_type=jnp.float32)
    h += g1_ref[...][:, 0:SPH] + g2_ref[...][:, SPH:C2]
    h = _silu(h + b1_ref[...])
    h = _silu(jnp.dot(h, w2_ref[...], preferred_element_type=jnp.float32) + b2_ref[...])
    rinv = 1.0 / (dist + 1e-12)
    nx, ny, nz = vx * rinv, vy * rinv, vz * rinv
    m = jnp.abs(ny) < 0.99
    b1x = jnp.where(m, nz, 0.0)
    b1y = jnp.where(m, 0.0, -nz)
    b1z = jnp.where(m, -nx, ny)
    bnorm = jnp.sqrt(b1x * b1x + b1y * b1y + b1z * b1z)
    bi = 1.0 / (bnorm + 1e-12)
    b1x, b1y, b1z = b1x * bi, b1y * bi, b1z * bi
    b2x = ny * b1z - nz * b1y
    b2y = nz * b1x - nx * b1z
    b2z = nx * b1y - ny * b1x
    zeros7 = jnp.zeros((BE, 7), jnp.float32)
    wig = jnp.concatenate(
        [ny, nz, nx, b2y, b2z, b2x, b1y, b1z, b1x, zeros7], axis=1)
    wig_ref[...] = wig
    env5 = env * (1.0 / 5.0)
    mw = [jnp.dot(h, w3p_ref[k], preferred_element_type=jnp.float32)
          for k in range(SPHSZ)]
    d = [[wig[:, 3 * i + j:3 * i + j + 1] for j in range(3)] for i in range(3)]
    mp = [None] * SPHSZ
    mp[0] = mw[0] * env5
    for i in range(3):
        acc = d[0][i] * mw[1] + d[1][i] * mw[2] + d[2][i] * mw[3]
        mp[1 + i] = acc * env5
    for k in range(4, SPHSZ):
        mp[k] = mw[k] * env5
    z = jnp.zeros((BE, SPH), jnp.float32)
    for p in range(PP):
        hi = mp[2 * p + 1] if 2 * p + 1 < SPHSZ else z
        msg0_ref[p] = jnp.concatenate([mp[2 * p], hi], axis=1)
    for l in range(NUM_LAYERS):
        gates_ref[l] = _silu(jnp.dot(h, gw_ref[l], preferred_element_type=jnp.float32)) * env


def _edge_prep(ev, g1, g2, w1d, b1, w2, b2, w3p, gw, e0=0, n_e=N_EDGES):
    grid = (n_e // BE,)
    ob = e0 // BE
    full = lambda i: (0, 0)
    full3 = lambda i: (0, 0, 0)
    return pl.pallas_call(
        _edge_prep_body,
        grid=grid,
        in_specs=[
            pl.BlockSpec((BE, 3), lambda i: (i + ob, 0)),
            pl.BlockSpec((BE, C2), lambda i: (i, 0)),
            pl.BlockSpec((BE, C2), lambda i: (i, 0)),
            pl.BlockSpec((NDB, EDGE_CH), full),
            pl.BlockSpec((1, EDGE_CH), full),
            pl.BlockSpec((EDGE_CH, EDGE_CH), full),
            pl.BlockSpec((1, EDGE_CH), full),
            pl.BlockSpec((SPHSZ, EDGE_CH, SPH), full3),
            pl.BlockSpec((NUM_LAYERS, EDGE_CH, SPH), full3),
        ],
        out_specs=[
            pl.BlockSpec((PP, BE, C2), lambda i: (0, i, 0)),
            pl.BlockSpec((NUM_LAYERS, BE, SPH), lambda i: (0, i, 0)),
            pl.BlockSpec((BE, 16), lambda i: (i, 0)),
        ],
        out_shape=[
            jax.ShapeDtypeStruct((PP, n_e, C2), jnp.float32),
            jax.ShapeDtypeStruct((NUM_LAYERS, n_e, SPH), jnp.float32),
            jax.ShapeDtypeStruct((n_e, 16), jnp.float32),
        ],
        interpret=INTERPRET,
    )(ev, g1, g2, w1d, b1, w2, b2, w3p, gw)


def _edge_layer_body(xs_ref, gate_ref, wig_ref, mw_ref, msg_ref):
    wig = wig_ref[...]
    d = [[wig[:, 3 * i + j:3 * i + j + 1] for j in range(3)] for i in range(3)]
    g = gate_ref[...]
    mw = mw_ref[...]
    def plane(k):
        h = (k % 2) * SPH
        return xs_ref[k // 2][:, h:h + SPH]
    def mm(v):
        return jnp.dot(v, mw, preferred_element_type=jnp.float32)
    m = [None] * SPHSZ
    m[0] = mm(plane(0) * g)
    xs1, xs2, xs3 = plane(1), plane(2), plane(3)
    mrot = []
    for i in range(3):
        y = d[i][0] * xs1 + d[i][1] * xs2 + d[i][2] * xs3
        mrot.append(mm(y * g))
    for i in range(3):
        m[1 + i] = d[0][i] * mrot[0] + d[1][i] * mrot[1] + d[2][i] * mrot[2]
    for k in range(4, SPHSZ):
        m[k] = mm(plane(k) * g)
    z = jnp.zeros((BE, SPH), jnp.float32)
    for p in range(PP):
        hi = m[2 * p + 1] if 2 * p + 1 < SPHSZ else z
        msg_ref[p] = jnp.concatenate([m[2 * p], hi], axis=1)


def _edge_layer(xs, gate, wig, mw):
    n_e = int(xs.shape[1])
    grid = (n_e // BE,)
    return pl.pallas_call(
        _edge_layer_body,
        grid=grid,
        in_specs=[
            pl.BlockSpec((PP, BE, C2), lambda i: (0, i, 0)),
            pl.BlockSpec((BE, SPH), lambda i: (i, 0)),
            pl.BlockSpec((BE, 16), lambda i: (i, 0)),
            pl.BlockSpec((SPH, SPH), lambda i: (0, 0)),
        ],
        out_specs=pl.BlockSpec((PP, BE, C2), lambda i: (0, i, 0)),
        out_shape=jax.ShapeDtypeStruct((PP, n_e, C2), jnp.float32),
        interpret=INTERPRET,
    )(xs, gate, wig, mw)


def _node_layer_body(xacc_ref, sysn_ref, w1_ref, w2_ref, out_ref):
    w1 = w1_ref[...]
    w2 = w2_ref[...]
    r = [None] * SPHSZ
    for k in range(SPHSZ):
        h = (k % 2) * SPH
        p = xacc_ref[k // 2][:, h:h + SPH]
        if k == 0:
            p = p + sysn_ref[...]
        mean = jnp.mean(p * p, axis=1, keepdims=True)
        pn = p * jax.lax.rsqrt(mean + 1e-6)
        ff = jnp.dot(_silu(jnp.dot(pn, w1, preferred_element_type=jnp.float32)),
                     w2, preferred_element_type=jnp.float32)
        r[k] = pn + ff
    z = jnp.zeros((BN, SPH), jnp.float32)
    for p in range(PP):
        hi = r[2 * p + 1] if 2 * p + 1 < SPHSZ else z
        out_ref[p] = jnp.concatenate([r[2 * p], hi], axis=1)


def _node_layer(xacc, sysn, w1, w2):
    grid = (NP // BN,)
    return pl.pallas_call(
        _node_layer_body,
        grid=grid,
        in_specs=[
            pl.BlockSpec((PP, BN, C2), lambda i: (0, i, 0)),
            pl.BlockSpec((BN, SPH), lambda i: (i, 0)),
            pl.BlockSpec((SPH, SPH), lambda i: (0, 0)),
            pl.BlockSpec((SPH, SPH), lambda i: (0, 0)),
        ],
        out_specs=pl.BlockSpec((PP, BN, C2), lambda i: (0, i, 0)),
        out_shape=jax.ShapeDtypeStruct((PP, NP, C2), jnp.float32),
        interpret=INTERPRET,
    )(xacc, sysn, w1, w2)


CORES = 2
TILES = 16
WORKERS = CORES * TILES
WPT = N_EDGES // WORKERS
GCH = 104
GK = 8
GGROUPS = WPT // (GK * GCH)
GTAIL = WPT - GGROUPS * GK * GCH
SEPT = N_EDGES // TILES
SCH = 104
SK = 3
SGROUPS = SEPT // (SK * SCH)
STAIL = SEPT - SGROUPS * SK * SCH
ROWS_PT = NP // TILES


GTAILMAX = 8


def _sc_gather(tables, idx, nplanes, e0=0, n_e=N_EDGES):
    from jax.experimental.pallas import tpu_sc as plsc
    P = nplanes
    shared_tab = tables.ndim == 2
    w_avg = n_e // WORKERS
    if w_avg % 8 == 0:
        variants = [(None, w_avg, w_avg)]
    else:
        w_lo = w_avg - (w_avg % 8)
        w_hi = 2 * w_avg - w_lo
        assert 16 * (w_lo + w_hi) == n_e and w_hi % 8 == 0
        variants = [("lo", w_lo, w_lo), ("hi", w_hi, w_lo)]
    mesh = plsc.VectorSubcoreMesh(core_axis_name="c", subcore_axis_name="s")

    @functools.partial(
        pl.kernel, mesh=mesh,
        out_type=jax.ShapeDtypeStruct((P, n_e, C2), jnp.float32),
        scratch_types=[
            pltpu.VMEM((GK * GCH,), jnp.int32),
            pltpu.VMEM((GK, GCH, C2), jnp.float32),
            pltpu.VMEM((GTAILMAX,), jnp.int32),
            pltpu.VMEM((GTAILMAX, C2), jnp.float32),
            pltpu.SemaphoreType.DMA,
            pltpu.SemaphoreType.DMA,
        ])
    def k(tab_h, idx_h, out_h, idx_v, rows_v, idxt_v, rowst_v, semg, semo):
        c = lax.axis_index("c")
        s = lax.axis_index("s")
        wid = s * CORES + c
        for kind, W, stride_lo in variants:
            ngroups = W // (GK * GCH)
            nleft = W // GCH - ngroups * GK
            tail = W - (W // GCH) * GCH
            if kind is None:
                pred = None
                wbase = pl.multiple_of(wid * W, 8)
            elif kind == "lo":
                pred = wid < 16
                wbase = pl.multiple_of(wid * stride_lo, 8)
            else:
                pred = wid >= 16
                wbase = pl.multiple_of(16 * stride_lo + (wid - 16) * W, 8)

            def emit(wbase=wbase, W=W, ngroups=ngroups, nleft=nleft, tail=tail):
                for p in range(P):
                    src = tab_h if shared_tab else tab_h.at[p]
                    ioff = (p * N_EDGES if shared_tab else 0) + e0

                    def chunk_group(gbase, nch, src=src, ioff=ioff, p=p):
                        pltpu.sync_copy(
                            idx_h.at[pl.ds(ioff + gbase, nch * GCH)],
                            idx_v.at[pl.ds(0, nch * GCH)])
                        hg = [pltpu.async_copy(
                                  src.at[idx_v.at[pl.ds(b * GCH, GCH)]],
                                  rows_v.at[b], semg)
                              for b in range(nch)]
                        ho = []
                        for b in range(nch):
                            hg[b].wait()
                            ob = pl.multiple_of(gbase + b * GCH, 8)
                            ho.append(pltpu.async_copy(
                                rows_v.at[b], out_h.at[p, pl.ds(ob, GCH)],
                                semo))
                        for h in ho:
                            h.wait()

                    def body(gi, carry):
                        chunk_group(pl.multiple_of(wbase + gi * (GK * GCH), 8),
                                    GK)
                        return carry
                    lax.fori_loop(0, ngroups, body, 0)
                    if nleft:
                        chunk_group(
                            pl.multiple_of(wbase + ngroups * GK * GCH, 8),
                            nleft)
                    if tail:
                        tbase = pl.multiple_of(wbase + (W // GCH) * GCH, 8)
                        pltpu.sync_copy(idx_h.at[pl.ds(ioff + tbase, tail)],
                                        idxt_v.at[pl.ds(0, tail)])
                        pltpu.async_copy(src.at[idxt_v.at[pl.ds(0, tail)]],
                                         rowst_v.at[pl.ds(0, tail)],
                                         semg).wait()
                        pltpu.sync_copy(rowst_v.at[pl.ds(0, tail)],
                                        out_h.at[p, pl.ds(tbase, tail)])

            if pred is None:
                emit()
            else:
                pl.when(pred)(emit)

    return k(tables, idx)


def _sc_scatter(msgs, idx, init, nplanes):
    from jax.experimental.pallas import tpu_sc as plsc
    P = nplanes
    nmsg = len(msgs)
    ns = [int(m.shape[1]) for m in msgs]
    offs = [0]
    for n in ns[:-1]:
        offs.append(offs[-1] + n)
    mesh = plsc.VectorSubcoreMesh(core_axis_name="c", subcore_axis_name="s")

    @functools.partial(
        pl.kernel, mesh=mesh,
        out_type=jax.ShapeDtypeStruct((P, NP, C2), jnp.float32),
        scratch_types=[
            pltpu.VMEM((SK, SCH), jnp.int32),
            pltpu.VMEM((SK, SCH, C2), jnp.float32),
            pltpu.VMEM((2, 8), jnp.int32),
            pltpu.VMEM((2, 8, C2), jnp.float32),
            pltpu.VMEM_SHARED((NP, C2), jnp.float32),
            pltpu.SemaphoreType.DMA,
        ])
    def k(*refs):
        msg_hs = refs[:nmsg]
        idx_h, init_h, out_h = refs[nmsg:nmsg + 3]
        idx2_v, m_v, idxt_v, mt_v, acc, sem = refs[nmsg + 3:]
        c = lax.axis_index("c")
        s = lax.axis_index("s")
        nb = pl.multiple_of(s * ROWS_PT, 8)
        for p in range(P):
            @pl.when(c == (p % CORES))
            def _(p=p):
                pltpu.sync_copy(init_h.at[p, pl.ds(nb, ROWS_PT)],
                                acc.at[pl.ds(nb, ROWS_PT)])
                plsc.subcore_barrier()
                for msg_h, e0, n_i in zip(msg_hs, offs, ns):
                    w_s = n_i // TILES
                    nch = w_s // SCH
                    ngroups = nch // SK
                    tail = w_s - nch * SCH
                    def group(gbase, k_in_flight, p=p, msg_h=msg_h, e0=e0):
                        hm = []
                        for b in range(k_in_flight):
                            ob = pl.multiple_of(gbase + b * SCH, 8)
                            pltpu.sync_copy(idx_h.at[pl.ds(e0 + ob, SCH)],
                                            idx2_v.at[b])
                            hm.append(pltpu.async_copy(
                                msg_h.at[p, pl.ds(ob, SCH)], m_v.at[b], sem))
                        for b in range(k_in_flight):
                            hm[b].wait()
                            pltpu.sync_copy(m_v.at[b], acc.at[idx2_v.at[b]],
                                            add=True)
                    def body(gi, carry, w_s=w_s):
                        group(pl.multiple_of(s * w_s + gi * (SK * SCH), 8), SK)
                        return carry
                    lax.fori_loop(0, ngroups, body, 0)
                    nleft = nch - ngroups * SK
                    if nleft:
                        group(pl.multiple_of(s * w_s + ngroups * SK * SCH, 8),
                              nleft)
                    for t in range(tail // 8):
                        baset = pl.multiple_of(s * w_s + nch * SCH + t * 8, 8)
                        pltpu.sync_copy(idx_h.at[pl.ds(e0 + baset, 8)],
                                        idxt_v.at[t])
                        pltpu.sync_copy(msg_h.at[p, pl.ds(baset, 8)],
                                        mt_v.at[t])
                        pltpu.sync_copy(mt_v.at[t], acc.at[idxt_v.at[t]],
                                        add=True)
                plsc.subcore_barrier()
                pltpu.sync_copy(acc.at[pl.ds(nb, ROWS_PT)],
                                out_h.at[p, pl.ds(nb, ROWS_PT)])
                plsc.subcore_barrier()

    return k(*msgs, idx, init)


def kernel(positions, atomic_numbers, batch, edge_index, edge_distance_vec,
           charge, spin, sphere_table, source_table, target_table, mix_w,
           mix_b, rad_w1, rad_b1, rad_w2, rad_b2, rad_w3, blk_gate_w,
           blk_msg_w, blk_ff_w1, blk_ff_w2):
    f32 = jnp.float32
    ei0 = edge_index[0].astype(jnp.int32)
    ei1 = edge_index[1].astype(jnp.int32)

    half = SPH // 2
    freqs2 = jnp.exp(jnp.arange(half, dtype=f32)
                     * (-np.log(10000.0) / half)).reshape(1, half)
    anum2 = jnp.pad(atomic_numbers.astype(jnp.int32), (0, NP - N_ATOMS)).reshape(NP, 1)
    batch2 = jnp.pad(batch.astype(jnp.int32), (0, NP - N_ATOMS)).reshape(NP, 1)
    charge2 = charge.astype(f32).reshape(N_SYS, 1)
    spin2 = spin.astype(f32).reshape(N_SYS, 1)
    pad_tab = lambda t: jnp.pad(t.astype(f32), ((0, 128 - MAX_ELEM), (0, 0)))
    sphere_p = pad_tab(sphere_table)
    src_p = pad_tab(source_table)
    tgt_p = pad_tab(target_table)
    mixb2 = mix_b.astype(f32).reshape(1, SPH)

    coeffs = [(l, m) for l in range(3) for m in range(-l, l + 1)]
    order = sorted(range(len(coeffs)),
                   key=lambda i: (abs(coeffs[i][1]), coeffs[i][1] < 0, coeffs[i][0]))
    P = np.zeros((SPHSZ, SPHSZ), dtype=np.float32)
    for row, col in enumerate(order):
        P[row, col] = 1.0
    permT = np.argmax(P, axis=0)
    w3p = (rad_w3.astype(f32).reshape(EDGE_CH, SPHSZ, SPH)[:, permT, :]
           .transpose(1, 0, 2))

    w1 = rad_w1.astype(f32)
    w1d, w1s, w1t = w1[:NDB], w1[NDB:NDB + EDGE_CH], w1[NDB + EDGE_CH:]
    b1 = (rad_b1.astype(f32)
          - 0.001 * (w1s.sum(axis=0) + w1t.sum(axis=0))).reshape(1, EDGE_CH)
    b2 = rad_b2.astype(f32).reshape(1, EDGE_CH)
    stw = jnp.concatenate([jnp.dot(src_p, w1s), jnp.dot(tgt_p, w1t)],
                          axis=1)

    x0row, sysn, nst = _node_prep(anum2, batch2, charge2, spin2,
                                  freqs2, mix_w.astype(f32), mixb2,
                                  sphere_p, stw)

    EH = N_EDGES // 2
    st_idx = jnp.concatenate([ei0, ei1])
    ev = edge_distance_vec.astype(f32)
    gw = blk_gate_w.astype(f32)
    w2 = rad_w2.astype(f32)
    g12A = _sc_gather(nst, st_idx, 2, 0, EH)
    msg0A, gatesA, wigA = _edge_prep(ev, g12A[0], g12A[1], w1d, b1, w2, b2,
                                     w3p, gw, 0, EH)
    g12B = _sc_gather(nst, st_idx, 2, EH, EH)
    msg0B, gatesB, wigB = _edge_prep(ev, g12B[0], g12B[1], w1d, b1, w2, b2,
                                     w3p, gw, EH, EH)

    init = jnp.zeros((PP, NP, C2), f32).at[0, :, 0:SPH].set(x0row)
    x = _sc_scatter([msg0A, msg0B], ei1, init, PP)

    for l in range(NUM_LAYERS):
        mwl = blk_msg_w[l].astype(f32)
        xsA = _sc_gather(x, ei0, PP, 0, EH)
        msgA = _edge_layer(xsA, gatesA[l], wigA, mwl)
        xsB = _sc_gather(x, ei0, PP, EH, EH)
        msgB = _edge_layer(xsB, gatesB[l], wigB, mwl)
        xacc = _sc_scatter([msgA, msgB], ei1, x, PP)
        x = _node_layer(xacc, sysn, blk_ff_w1[l].astype(f32),
                        blk_ff_w2[l].astype(f32))

    planes = (x.reshape(PP, NP, 2, SPH).transpose(0, 2, 1, 3)
              .reshape(2 * PP, NP, SPH)[:SPHSZ])
    return planes[:, :N_ATOMS].transpose(1, 0, 2)


# speedup vs baseline: 1.0240x; 1.0240x over previous
import functools

import jax
import jax.numpy as jnp
import numpy as np
from jax import lax
from jax.experimental import pallas as pl
from jax.experimental.pallas import tpu as pltpu

N_ATOMS = 10000
NP = 10240
N_EDGES = 160000
N_SYS = 16
SPH = 64
SPHSZ = 9
PP = 5
C2 = 128
NDB = 256
EDGE_CH = 64
CUTOFF = 5.0
NUM_LAYERS = 2
MAX_ELEM = 100

BN = 1024
BE = 800

INTERPRET = False


def _silu(x):
    return x * jax.nn.sigmoid(x)


def _node_prep_body(anum_ref, batch_ref, charge_ref, spin_ref, freqs_ref,
                    mixw_ref, mixb_ref, sphere_ref, stw_ref,
                    x0_ref, sysn_ref, nst_ref):
    chg = charge_ref[...]
    spn = spin_ref[...]
    freqs = freqs_ref[...]
    angc = chg * freqs
    angs = spn * freqs
    mw = mixw_ref[...]
    csd = (jnp.dot(jnp.sin(angc), mw[0:32], preferred_element_type=jnp.float32)
           + jnp.dot(jnp.cos(angc), mw[32:64], preferred_element_type=jnp.float32)
           + jnp.dot(jnp.sin(angs), mw[64:96], preferred_element_type=jnp.float32)
           + jnp.dot(jnp.cos(angs), mw[96:128], preferred_element_type=jnp.float32))
    csd = _silu(csd + mixb_ref[...])
    anum = anum_ref[...]
    oh_a = (anum == lax.broadcasted_iota(jnp.int32, (BN, 128), 1)).astype(jnp.float32)
    batch = batch_ref[...]
    oh_b = (batch == lax.broadcasted_iota(jnp.int32, (BN, 16), 1)).astype(jnp.float32)
    atom_emb = jnp.dot(oh_a, sphere_ref[...], preferred_element_type=jnp.float32)
    sys_node = jnp.dot(oh_b, csd, preferred_element_type=jnp.float32)
    x0_ref[...] = atom_emb + sys_node
    sysn_ref[...] = sys_node
    nst_ref[...] = jnp.dot(oh_a, stw_ref[...], preferred_element_type=jnp.float32)


def _node_prep(anum2, batch2, charge2, spin2, freqs2, mixw, mixb, sphere_p, stw):
    grid = (NP // BN,)
    full = lambda i: (0, 0)
    return pl.pallas_call(
        _node_prep_body,
        grid=grid,
        in_specs=[
            pl.BlockSpec((BN, 1), lambda i: (i, 0)),
            pl.BlockSpec((BN, 1), lambda i: (i, 0)),
            pl.BlockSpec((N_SYS, 1), full),
            pl.BlockSpec((N_SYS, 1), full),
            pl.BlockSpec((1, 32), full),
            pl.BlockSpec((128, SPH), full),
            pl.BlockSpec((1, SPH), full),
            pl.BlockSpec((128, SPH), full),
            pl.BlockSpec((128, C2), full),
        ],
        out_specs=[
            pl.BlockSpec((BN, SPH), lambda i: (i, 0)),
            pl.BlockSpec((BN, SPH), lambda i: (i, 0)),
            pl.BlockSpec((BN, C2), lambda i: (i, 0)),
        ],
        out_shape=[
            jax.ShapeDtypeStruct((NP, SPH), jnp.float32),
            jax.ShapeDtypeStruct((NP, SPH), jnp.float32),
            jax.ShapeDtypeStruct((NP, C2), jnp.float32),
        ],
        interpret=INTERPRET,
    )(anum2, batch2, charge2, spin2, freqs2, mixw, mixb, sphere_p, stw)


def _edge_prep_body(ev_ref, g1_ref, g2_ref, w1d_ref, b1_ref,
                    w2_ref, b2_ref, w3p_ref, gw_ref,
                    msg0_ref, gates_ref, wig_ref):
    ev = ev_ref[...]
    vx, vy, vz = ev[:, 0:1], ev[:, 1:2], ev[:, 2:3]
    d2 = vx * vx + vy * vy + vz * vz
    dist = jnp.sqrt(d2)
    u = dist * (1.0 / CUTOFF)
    u2 = u * u
    u4 = u2 * u2
    u5 = u4 * u
    env = 1.0 + (-21.0) * u5 + 35.0 * u5 * u + (-15.0) * u5 * u2
    env = jnp.where(u < 1.0, env, 0.0)
    step = CUTOFF / (NDB - 1)
    coeff = -0.5 / (2.0 * step) ** 2
    offs = lax.broadcasted_iota(jnp.int32, (BE, NDB), 1).astype(jnp.float32) * step
    diff = dist - offs
    demb = jnp.exp(coeff * diff * diff)
    h = jnp.dot(demb, w1d_ref[...], preferred_element_type=jnp.float32)
    h += g1_ref[...][:, 0:SPH] + g2_ref[...][:, SPH:C2]
    h = _silu(h + b1_ref[...])
    h = _silu(jnp.dot(h, w2_ref[...], preferred_element_type=jnp.float32) + b2_ref[...])
    rinv = 1.0 / (dist + 1e-12)
    nx, ny, nz = vx * rinv, vy * rinv, vz * rinv
    m = jnp.abs(ny) < 0.99
    b1x = jnp.where(m, nz, 0.0)
    b1y = jnp.where(m, 0.0, -nz)
    b1z = jnp.where(m, -nx, ny)
    bnorm = jnp.sqrt(b1x * b1x + b1y * b1y + b1z * b1z)
    bi = 1.0 / (bnorm + 1e-12)
    b1x, b1y, b1z = b1x * bi, b1y * bi, b1z * bi
    b2x = ny * b1z - nz * b1y
    b2y = nz * b1x - nx * b1z
    b2z = nx * b1y - ny * b1x
    zeros7 = jnp.zeros((BE, 7), jnp.float32)
    wig = jnp.concatenate(
        [ny, nz, nx, b2y, b2z, b2x, b1y, b1z, b1x, zeros7], axis=1)
    wig_ref[...] = wig
    env5 = env * (1.0 / 5.0)
    mw = [jnp.dot(h, w3p_ref[k], preferred_element_type=jnp.float32)
          for k in range(SPHSZ)]
    d = [[wig[:, 3 * i + j:3 * i + j + 1] for j in range(3)] for i in range(3)]
    mp = [None] * SPHSZ
    mp[0] = mw[0] * env5
    for i in range(3):
        acc = d[0][i] * mw[1] + d[1][i] * mw[2] + d[2][i] * mw[3]
        mp[1 + i] = acc * env5
    for k in range(4, SPHSZ):
        mp[k] = mw[k] * env5
    z = jnp.zeros((BE, SPH), jnp.float32)
    for p in range(PP):
        hi = mp[2 * p + 1] if 2 * p + 1 < SPHSZ else z
        msg0_ref[p] = jnp.concatenate([mp[2 * p], hi], axis=1)
    for l in range(NUM_LAYERS):
        gates_ref[l] = _silu(jnp.dot(h, gw_ref[l], preferred_element_type=jnp.float32)) * env


def _edge_prep(ev, g1, g2, w1d, b1, w2, b2, w3p, gw, e0=0, n_e=N_EDGES):
    grid = (n_e // BE,)
    ob = e0 // BE
    full = lambda i: (0, 0)
    full3 = lambda i: (0, 0, 0)
    return pl.pallas_call(
        _edge_prep_body,
        grid=grid,
        in_specs=[
            pl.BlockSpec((BE, 3), lambda i: (i + ob, 0)),
            pl.BlockSpec((BE, C2), lambda i: (i, 0)),
            pl.BlockSpec((BE, C2), lambda i: (i, 0)),
            pl.BlockSpec((NDB, EDGE_CH), full),
            pl.BlockSpec((1, EDGE_CH), full),
            pl.BlockSpec((EDGE_CH, EDGE_CH), full),
            pl.BlockSpec((1, EDGE_CH), full),
            pl.BlockSpec((SPHSZ, EDGE_CH, SPH), full3),
            pl.BlockSpec((NUM_LAYERS, EDGE_CH, SPH), full3),
        ],
        out_specs=[
            pl.BlockSpec((PP, BE, C2), lambda i: (0, i, 0)),
            pl.BlockSpec((NUM_LAYERS, BE, SPH), lambda i: (0, i, 0)),
            pl.BlockSpec((BE, 16), lambda i: (i, 0)),
        ],
        out_shape=[
            jax.ShapeDtypeStruct((PP, n_e, C2), jnp.float32),
            jax.ShapeDtypeStruct((NUM_LAYERS, n_e, SPH), jnp.float32),
            jax.ShapeDtypeStruct((n_e, 16), jnp.float32),
        ],
        interpret=INTERPRET,
    )(ev, g1, g2, w1d, b1, w2, b2, w3p, gw)


def _edge_layer_body(xs_ref, gate_ref, wig_ref, mw_ref, msg_ref):
    wig = wig_ref[...]
    d = [[wig[:, 3 * i + j:3 * i + j + 1] for j in range(3)] for i in range(3)]
    g = gate_ref[...]
    mw = mw_ref[...]
    def plane(k):
        h = (k % 2) * SPH
        return xs_ref[k // 2][:, h:h + SPH]
    def mm(v):
        return jnp.dot(v, mw, preferred_element_type=jnp.float32)
    m = [None] * SPHSZ
    m[0] = mm(plane(0) * g)
    xs1, xs2, xs3 = plane(1), plane(2), plane(3)
    mrot = []
    for i in range(3):
        y = d[i][0] * xs1 + d[i][1] * xs2 + d[i][2] * xs3
        mrot.append(mm(y * g))
    for i in range(3):
        m[1 + i] = d[0][i] * mrot[0] + d[1][i] * mrot[1] + d[2][i] * mrot[2]
    for k in range(4, SPHSZ):
        m[k] = mm(plane(k) * g)
    z = jnp.zeros((BE, SPH), jnp.float32)
    for p in range(PP):
        hi = m[2 * p + 1] if 2 * p + 1 < SPHSZ else z
        msg_ref[p] = jnp.concatenate([m[2 * p], hi], axis=1)


def _edge_layer(xs, gate, wig, mw):
    n_e = int(xs.shape[1])
    grid = (n_e // BE,)
    return pl.pallas_call(
        _edge_layer_body,
        grid=grid,
        in_specs=[
            pl.BlockSpec((PP, BE, C2), lambda i: (0, i, 0)),
            pl.BlockSpec((BE, SPH), lambda i: (i, 0)),
            pl.BlockSpec((BE, 16), lambda i: (i, 0)),
            pl.BlockSpec((SPH, SPH), lambda i: (0, 0)),
        ],
        out_specs=pl.BlockSpec((PP, BE, C2), lambda i: (0, i, 0)),
        out_shape=jax.ShapeDtypeStruct((PP, n_e, C2), jnp.float32),
        interpret=INTERPRET,
    )(xs, gate, wig, mw)


def _node_layer_body(xacc_ref, sysn_ref, w1_ref, w2_ref, out_ref):
    w1 = w1_ref[...]
    w2 = w2_ref[...]
    split_last = xacc_ref.shape[0] == PP + 1
    r = [None] * SPHSZ
    for k in range(SPHSZ):
        h = (k % 2) * SPH
        if split_last and k == SPHSZ - 1:
            p = xacc_ref[PP - 1][:, h:h + SPH] + xacc_ref[PP][:, h:h + SPH]
        else:
            p = xacc_ref[k // 2][:, h:h + SPH]
        if k == 0:
            p = p + sysn_ref[...]
        mean = jnp.mean(p * p, axis=1, keepdims=True)
        pn = p * jax.lax.rsqrt(mean + 1e-6)
        ff = jnp.dot(_silu(jnp.dot(pn, w1, preferred_element_type=jnp.float32)),
                     w2, preferred_element_type=jnp.float32)
        r[k] = pn + ff
    z = jnp.zeros((BN, SPH), jnp.float32)
    for p in range(PP):
        hi = r[2 * p + 1] if 2 * p + 1 < SPHSZ else z
        out_ref[p] = jnp.concatenate([r[2 * p], hi], axis=1)


def _node_layer(xacc, sysn, w1, w2):
    grid = (NP // BN,)
    nin = int(xacc.shape[0])
    return pl.pallas_call(
        _node_layer_body,
        grid=grid,
        in_specs=[
            pl.BlockSpec((nin, BN, C2), lambda i: (0, i, 0)),
            pl.BlockSpec((BN, SPH), lambda i: (i, 0)),
            pl.BlockSpec((SPH, SPH), lambda i: (0, 0)),
            pl.BlockSpec((SPH, SPH), lambda i: (0, 0)),
        ],
        out_specs=pl.BlockSpec((PP, BN, C2), lambda i: (0, i, 0)),
        out_shape=jax.ShapeDtypeStruct((PP, NP, C2), jnp.float32),
        interpret=INTERPRET,
    )(xacc, sysn, w1, w2)


CORES = 2
TILES = 16
WORKERS = CORES * TILES
WPT = N_EDGES // WORKERS
GCH = 104
GK = 8
GGROUPS = WPT // (GK * GCH)
GTAIL = WPT - GGROUPS * GK * GCH
SEPT = N_EDGES // TILES
SCH = 104
SK = 3
SGROUPS = SEPT // (SK * SCH)
STAIL = SEPT - SGROUPS * SK * SCH
ROWS_PT = NP // TILES


GTAILMAX = 8


def _sc_gather(tables, idx, nplanes, e0=0, n_e=N_EDGES):
    from jax.experimental.pallas import tpu_sc as plsc
    P = nplanes
    shared_tab = tables.ndim == 2
    w_avg = n_e // WORKERS
    if w_avg % 8 == 0:
        variants = [(None, w_avg, w_avg)]
    else:
        w_lo = w_avg - (w_avg % 8)
        w_hi = 2 * w_avg - w_lo
        assert 16 * (w_lo + w_hi) == n_e and w_hi % 8 == 0
        variants = [("lo", w_lo, w_lo), ("hi", w_hi, w_lo)]
    mesh = plsc.VectorSubcoreMesh(core_axis_name="c", subcore_axis_name="s")

    @functools.partial(
        pl.kernel, mesh=mesh,
        out_type=jax.ShapeDtypeStruct((P, n_e, C2), jnp.float32),
        scratch_types=[
            pltpu.VMEM((GK * GCH,), jnp.int32),
            pltpu.VMEM((GK, GCH, C2), jnp.float32),
            pltpu.VMEM((GTAILMAX,), jnp.int32),
            pltpu.VMEM((GTAILMAX, C2), jnp.float32),
            pltpu.SemaphoreType.DMA,
            pltpu.SemaphoreType.DMA,
        ])
    def k(tab_h, idx_h, out_h, idx_v, rows_v, idxt_v, rowst_v, semg, semo):
        c = lax.axis_index("c")
        s = lax.axis_index("s")
        wid = s * CORES + c
        for kind, W, stride_lo in variants:
            ngroups = W // (GK * GCH)
            nleft = W // GCH - ngroups * GK
            tail = W - (W // GCH) * GCH
            if kind is None:
                pred = None
                wbase = pl.multiple_of(wid * W, 8)
            elif kind == "lo":
                pred = wid < 16
                wbase = pl.multiple_of(wid * stride_lo, 8)
            else:
                pred = wid >= 16
                wbase = pl.multiple_of(16 * stride_lo + (wid - 16) * W, 8)

            def emit(wbase=wbase, W=W, ngroups=ngroups, nleft=nleft, tail=tail):
                for p in range(P):
                    src = tab_h if shared_tab else tab_h.at[p]
                    ioff = (p * N_EDGES if shared_tab else 0) + e0

                    def chunk_group(gbase, nch, src=src, ioff=ioff, p=p):
                        pltpu.sync_copy(
                            idx_h.at[pl.ds(ioff + gbase, nch * GCH)],
                            idx_v.at[pl.ds(0, nch * GCH)])
                        hg = [pltpu.async_copy(
                                  src.at[idx_v.at[pl.ds(b * GCH, GCH)]],
                                  rows_v.at[b], semg)
                              for b in range(nch)]
                        ho = []
                        for b in range(nch):
                            hg[b].wait()
                            ob = pl.multiple_of(gbase + b * GCH, 8)
                            ho.append(pltpu.async_copy(
                                rows_v.at[b], out_h.at[p, pl.ds(ob, GCH)],
                                semo))
                        for h in ho:
                            h.wait()

                    def body(gi, carry):
                        chunk_group(pl.multiple_of(wbase + gi * (GK * GCH), 8),
                                    GK)
                        return carry
                    lax.fori_loop(0, ngroups, body, 0)
                    if nleft:
                        chunk_group(
                            pl.multiple_of(wbase + ngroups * GK * GCH, 8),
                            nleft)
                    if tail:
                        tbase = pl.multiple_of(wbase + (W // GCH) * GCH, 8)
                        pltpu.sync_copy(idx_h.at[pl.ds(ioff + tbase, tail)],
                                        idxt_v.at[pl.ds(0, tail)])
                        pltpu.async_copy(src.at[idxt_v.at[pl.ds(0, tail)]],
                                         rowst_v.at[pl.ds(0, tail)],
                                         semg).wait()
                        pltpu.sync_copy(rowst_v.at[pl.ds(0, tail)],
                                        out_h.at[p, pl.ds(tbase, tail)])

            if pred is None:
                emit()
            else:
                pl.when(pred)(emit)

    return k(tables, idx)


def _sc_scatter(msgs, idx, init, nplanes, zeros=None):
    from jax.experimental.pallas import tpu_sc as plsc
    P = nplanes
    split_last = zeros is not None
    nmsg = len(msgs)
    assert not split_last or nmsg == 2
    ns = [int(m.shape[1]) for m in msgs]
    offs = [0]
    for n in ns[:-1]:
        offs.append(offs[-1] + n)
    n_out = P + 1 if split_last else P
    mesh = plsc.VectorSubcoreMesh(core_axis_name="c", subcore_axis_name="s")

    @functools.partial(
        pl.kernel, mesh=mesh,
        out_type=jax.ShapeDtypeStruct((n_out, NP, C2), jnp.float32),
        scratch_types=[
            pltpu.VMEM((SK, SCH), jnp.int32),
            pltpu.VMEM((SK, SCH, C2), jnp.float32),
            pltpu.VMEM((2, 8), jnp.int32),
            pltpu.VMEM((2, 8, C2), jnp.float32),
            pltpu.VMEM_SHARED((NP, C2), jnp.float32),
            pltpu.SemaphoreType.DMA,
        ])
    def k(*refs):
        msg_hs = refs[:nmsg]
        pos = nmsg
        if split_last:
            zeros_h = refs[pos]
            pos += 1
        idx_h, init_h, out_h = refs[pos:pos + 3]
        idx2_v, m_v, idxt_v, mt_v, acc, sem = refs[pos + 3:]
        c = lax.axis_index("c")
        s = lax.axis_index("s")
        nb = pl.multiple_of(s * ROWS_PT, 8)

        def add_source(msg_h, e0, n_i, p):
            w_s = n_i // TILES
            nch = w_s // SCH
            ngroups = nch // SK
            tail = w_s - nch * SCH
            def group(gbase, k_in_flight):
                hm = []
                for b in range(k_in_flight):
                    ob = pl.multiple_of(gbase + b * SCH, 8)
                    pltpu.sync_copy(idx_h.at[pl.ds(e0 + ob, SCH)],
                                    idx2_v.at[b])
                    hm.append(pltpu.async_copy(
                        msg_h.at[p, pl.ds(ob, SCH)], m_v.at[b], sem))
                for b in range(k_in_flight):
                    hm[b].wait()
                    pltpu.sync_copy(m_v.at[b], acc.at[idx2_v.at[b]],
                                    add=True)
            def body(gi, carry):
                group(pl.multiple_of(s * w_s + gi * (SK * SCH), 8), SK)
                return carry
            lax.fori_loop(0, ngroups, body, 0)
            nleft = nch - ngroups * SK
            if nleft:
                group(pl.multiple_of(s * w_s + ngroups * SK * SCH, 8), nleft)
            for t in range(tail // 8):
                baset = pl.multiple_of(s * w_s + nch * SCH + t * 8, 8)
                pltpu.sync_copy(idx_h.at[pl.ds(e0 + baset, 8)], idxt_v.at[t])
                pltpu.sync_copy(msg_h.at[p, pl.ds(baset, 8)], mt_v.at[t])
                pltpu.sync_copy(mt_v.at[t], acc.at[idxt_v.at[t]], add=True)

        n_whole = P - 1 if split_last else P
        for p in range(n_whole):
            @pl.when(c == (p % CORES))
            def _(p=p):
                pltpu.sync_copy(init_h.at[p, pl.ds(nb, ROWS_PT)],
                                acc.at[pl.ds(nb, ROWS_PT)])
                plsc.subcore_barrier()
                for msg_h, e0, n_i in zip(msg_hs, offs, ns):
                    add_source(msg_h, e0, n_i, p)
                plsc.subcore_barrier()
                pltpu.sync_copy(acc.at[pl.ds(nb, ROWS_PT)],
                                out_h.at[p, pl.ds(nb, ROWS_PT)])
                plsc.subcore_barrier()
        if split_last:
            p = P - 1
            for ci in range(CORES):
                @pl.when(c == ci)
                def _(ci=ci, p=p):
                    base_h = init_h if ci == 0 else zeros_h
                    src_slice = (base_h.at[p, pl.ds(nb, ROWS_PT)] if ci == 0
                                 else base_h.at[pl.ds(nb, ROWS_PT)])
                    pltpu.sync_copy(src_slice, acc.at[pl.ds(nb, ROWS_PT)])
                    plsc.subcore_barrier()
                    add_source(msg_hs[ci], offs[ci], ns[ci], p)
                    plsc.subcore_barrier()
                    pltpu.sync_copy(acc.at[pl.ds(nb, ROWS_PT)],
                                    out_h.at[p + ci, pl.ds(nb, ROWS_PT)])
                    plsc.subcore_barrier()

    args = list(msgs) + ([zeros] if split_last else []) + [idx, init]
    return k(*args)


def kernel(positions, atomic_numbers, batch, edge_index, edge_distance_vec,
           charge, spin, sphere_table, source_table, target_table, mix_w,
           mix_b, rad_w1, rad_b1, rad_w2, rad_b2, rad_w3, blk_gate_w,
           blk_msg_w, blk_ff_w1, blk_ff_w2):
    f32 = jnp.float32
    ei0 = edge_index[0].astype(jnp.int32)
    ei1 = edge_index[1].astype(jnp.int32)

    half = SPH // 2
    freqs2 = jnp.exp(jnp.arange(half, dtype=f32)
                     * (-np.log(10000.0) / half)).reshape(1, half)
    anum2 = jnp.pad(atomic_numbers.astype(jnp.int32), (0, NP - N_ATOMS)).reshape(NP, 1)
    batch2 = jnp.pad(batch.astype(jnp.int32), (0, NP - N_ATOMS)).reshape(NP, 1)
    charge2 = charge.astype(f32).reshape(N_SYS, 1)
    spin2 = spin.astype(f32).reshape(N_SYS, 1)
    pad_tab = lambda t: jnp.pad(t.astype(f32), ((0, 128 - MAX_ELEM), (0, 0)))
    sphere_p = pad_tab(sphere_table)
    src_p = pad_tab(source_table)
    tgt_p = pad_tab(target_table)
    mixb2 = mix_b.astype(f32).reshape(1, SPH)

    coeffs = [(l, m) for l in range(3) for m in range(-l, l + 1)]
    order = sorted(range(len(coeffs)),
                   key=lambda i: (abs(coeffs[i][1]), coeffs[i][1] < 0, coeffs[i][0]))
    P = np.zeros((SPHSZ, SPHSZ), dtype=np.float32)
    for row, col in enumerate(order):
        P[row, col] = 1.0
    permT = np.argmax(P, axis=0)
    w3p = (rad_w3.astype(f32).reshape(EDGE_CH, SPHSZ, SPH)[:, permT, :]
           .transpose(1, 0, 2))

    w1 = rad_w1.astype(f32)
    w1d, w1s, w1t = w1[:NDB], w1[NDB:NDB + EDGE_CH], w1[NDB + EDGE_CH:]
    b1 = (rad_b1.astype(f32)
          - 0.001 * (w1s.sum(axis=0) + w1t.sum(axis=0))).reshape(1, EDGE_CH)
    b2 = rad_b2.astype(f32).reshape(1, EDGE_CH)
    stw = jnp.concatenate([jnp.dot(src_p, w1s), jnp.dot(tgt_p, w1t)],
                          axis=1)

    x0row, sysn, nst = _node_prep(anum2, batch2, charge2, spin2,
                                  freqs2, mix_w.astype(f32), mixb2,
                                  sphere_p, stw)

    EH = N_EDGES // 2
    st_idx = jnp.concatenate([ei0, ei1])
    ev = edge_distance_vec.astype(f32)
    gw = blk_gate_w.astype(f32)
    w2 = rad_w2.astype(f32)
    g12A = _sc_gather(nst, st_idx, 2, 0, EH)
    msg0A, gatesA, wigA = _edge_prep(ev, g12A[0], g12A[1], w1d, b1, w2, b2,
                                     w3p, gw, 0, EH)
    g12B = _sc_gather(nst, st_idx, 2, EH, EH)
    msg0B, gatesB, wigB = _edge_prep(ev, g12B[0], g12B[1], w1d, b1, w2, b2,
                                     w3p, gw, EH, EH)

    init = jnp.zeros((PP, NP, C2), f32).at[0, :, 0:SPH].set(x0row)
    x = _sc_scatter([msg0A, msg0B], ei1, init, PP)

    zrows = jnp.zeros((NP, C2), f32)
    for l in range(NUM_LAYERS):
        mwl = blk_msg_w[l].astype(f32)
        xsA = _sc_gather(x, ei0, PP, 0, EH)
        msgA = _edge_layer(xsA, gatesA[l], wigA, mwl)
        xsB = _sc_gather(x, ei0, PP, EH, EH)
        msgB = _edge_layer(xsB, gatesB[l], wigB, mwl)
        xacc = _sc_scatter([msgA, msgB], ei1, x, PP, zeros=zrows)
        x = _node_layer(xacc, sysn, blk_ff_w1[l].astype(f32),
                        blk_ff_w2[l].astype(f32))

    planes = (x.reshape(PP, NP, 2, SPH).transpose(0, 2, 1, 3)
              .reshape(2 * PP, NP, SPH)[:SPHSZ])
    return planes[:, :N_ATOMS].transpose(1, 0, 2)
